# baseline (speedup 1.0000x reference)
import functools

import jax
import jax.numpy as jnp
from jax import lax
from jax.experimental import pallas as pl

_DOT_HI = functools.partial(lax.dot_general,
                            precision=lax.Precision.HIGHEST,
                            preferred_element_type=jnp.float32)
_DOT = functools.partial(lax.dot_general,
                         preferred_element_type=jnp.float32)


def _conv_pallas(patches, wmat, bias, epilogue='none'):
    N, K, L = patches.shape
    Co = wmat.shape[0]

    def body(p_ref, w_ref, b_ref, o_ref):
        z = _DOT(w_ref[...], p_ref[0], (((1,), (0,)), ((), ())))
        z = z + b_ref[...]
        if epilogue == 'tanh':
            z = jnp.tanh(z)
        o_ref[0] = z

    return pl.pallas_call(
        body, grid=(N,),
        in_specs=[
            pl.BlockSpec((1, K, L), lambda n: (n, 0, 0)),
            pl.BlockSpec((Co, K), lambda n: (0, 0)),
            pl.BlockSpec((Co, 1), lambda n: (0, 0)),
        ],
        out_specs=pl.BlockSpec((1, Co, L), lambda n: (n, 0, 0)),
        out_shape=jax.ShapeDtypeStruct((N, Co, L), jnp.float32))(
            patches, wmat, bias)


def _conv_block(x, wgt, b, stride, pad, epilogue='none'):
    N, Cin, H, W = x.shape
    Co, _, k, _ = wgt.shape
    xp = jnp.pad(x, ((0, 0), (0, 0), (pad, pad), (pad, pad)), mode='reflect')
    xp = xp.astype(jnp.bfloat16)
    Hp, Wp = H + 2 * pad, W + 2 * pad
    Ho, Wo = (Hp - k) // stride + 1, (Wp - k) // stride + 1
    taps = [xp[:, :, kh:kh + stride * Ho:stride, kw:kw + stride * Wo:stride]
            for kh in range(k) for kw in range(k)]
    patches = jnp.stack(taps, axis=1).reshape(N, k * k * Cin, Ho * Wo)
    wmat = jnp.transpose(wgt, (0, 2, 3, 1)).reshape(Co, k * k * Cin)
    z = _conv_pallas(patches, wmat.astype(jnp.bfloat16), b[:, None],
                     epilogue=epilogue)
    return z.reshape(N, Co, Ho, Wo)


def _bn_lrelu(z):
    z = lax.optimization_barrier(z)
    axes = (0, 2, 3) if z.ndim == 4 else (0, 2)
    m = jnp.mean(z, axis=axes, keepdims=True)
    v = jnp.var(z, axis=axes, keepdims=True)
    zn = (z - m) / jnp.sqrt(v + 1e-5)
    return lax.optimization_barrier(jnp.where(zn >= 0, zn, 0.2 * zn))


def _normalize(x, axis):
    n = jnp.linalg.norm(x, axis=axis, keepdims=True)
    return x / jnp.maximum(n, 1e-12)


def _fusion(feat, refs, wfs, wrefs, sim, fidx):
    B, c, hw = feat.shape
    n_refs = refs.shape[1]
    num = wfs.shape[2]

    def body(f_ref, r_ref, wq_ref, wr_ref, s_ref, x_ref, fg_ref, ri_ref):
        feat_v = f_ref[0]
        idx2 = x_ref[0]
        pio = lax.broadcasted_iota(jnp.int32, (hw, num), 0)
        oh = (pio == idx2).astype(jnp.float32)
        feat_sel = _DOT_HI(feat_v, oh, (((1,), (0,)), ((), ())))
        sims = s_ref[0]
        accum = feat_sel * sims[:, 0:1]
        wq16 = wq_ref[0].astype(jnp.bfloat16)
        inds = []
        for j in range(n_refs):
            refj = r_ref[0, j]
            wr16 = wr_ref[0, j].astype(jnp.bfloat16)
            fx = _DOT(wq16, wr16, (((0,), (0,)), ((), ())))
            ind = jnp.argmax(fx, axis=1).astype(jnp.int32)
            ohj = (pio == ind[None, :]).astype(jnp.float32)
            sel = _DOT_HI(refj, ohj, (((1,), (0,)), ((), ())))
            accum = accum + sel * sims[:, j + 1:j + 2]
            inds.append(ind[None, :])
        scat = _DOT_HI(accum, oh, (((1,), (1,)), ((), ())))
        maskv = jnp.sum(oh, axis=1)[None, :]
        fg_ref[0] = feat_v * (1.0 - maskv) + scat
        ri_ref[0] = jnp.concatenate(inds, axis=1)

    return pl.pallas_call(
        body, grid=(B,),
        in_specs=[
            pl.BlockSpec((1, c, hw), lambda b_: (b_, 0, 0)),
            pl.BlockSpec((1, n_refs, c, hw), lambda b_: (b_, 0, 0, 0)),
            pl.BlockSpec((1, c, num), lambda b_: (b_, 0, 0)),
            pl.BlockSpec((1, n_refs, c, hw), lambda b_: (b_, 0, 0, 0)),
            pl.BlockSpec((1, 1, sim.shape[2]), lambda b_: (b_, 0, 0)),
            pl.BlockSpec((1, 1, num), lambda b_: (b_, 0, 0)),
        ],
        out_specs=[
            pl.BlockSpec((1, c, hw), lambda b_: (b_, 0, 0)),
            pl.BlockSpec((1, 1, n_refs * num), lambda b_: (b_, 0, 0)),
        ],
        out_shape=[
            jax.ShapeDtypeStruct((B, c, hw), jnp.float32),
            jax.ShapeDtypeStruct((B, 1, n_refs * num), jnp.int32),
        ])(feat, refs, wfs, wrefs, sim, fidx)


def kernel(xs, params):
    B, K, C, H, W = xs.shape
    enc, dec = params['enc'], params['dec']
    x = xs.reshape(B * K, C, H, W)

    for i, ((wgt, b), (s, p)) in enumerate(zip(
            enc, [(1, 2), (2, 1), (2, 1), (2, 1), (2, 1)])):
        if i == 4:
            xp = jnp.pad(x, ((0, 0), (0, 0), (p, p), (p, p)),
                         mode='reflect')
            z = lax.conv_general_dilated(
                xp, wgt, (s, s), 'VALID',
                dimension_numbers=('NCHW', 'OIHW', 'NCHW'))
            z = z + b[None, :, None, None]
            x = _bn_lrelu(z)
        else:
            x = _bn_lrelu(_conv_block(x, wgt, b, s, p))

    c, h, wd = x.shape[1], x.shape[2], x.shape[3]
    hw = h * wd
    num = hw // 2
    n_refs = K - 1
    querys = x.reshape(B, K, c, hw)

    rk = jax.random.key(42)
    sim = jax.random.uniform(jax.random.fold_in(rk, 0), (B, K),
                             dtype=jnp.float32)
    sim = sim / jnp.sum(sim, axis=1, keepdims=True)
    perm_keys = jax.random.split(jax.random.fold_in(rk, 1), B)
    feat_indices = jnp.stack(
        [jax.random.permutation(perm_keys[i], hw)[:num] for i in range(B)],
        axis=0)

    querys = lax.optimization_barrier(querys)
    feat_flat = querys[:, 0]
    refs_flat = querys[:, 1:]
    w_feat = _normalize(jnp.transpose(feat_flat, (0, 2, 1)), 2)
    w_refs = _normalize(
        jnp.transpose(refs_flat, (0, 2, 1, 3)).reshape(B, c, n_refs * hw),
        1).reshape(B, c, n_refs, hw)
    w_feat_select = jnp.take_along_axis(
        w_feat, feat_indices[:, :, None], axis=1)
    w_feat_select = _normalize(w_feat_select, 2)

    w_feat_select, w_refs = lax.optimization_barrier((w_feat_select, w_refs))
    fg, ridx = _fusion(
        feat_flat, refs_flat,
        jnp.transpose(w_feat_select, (0, 2, 1)),
        jnp.transpose(w_refs, (0, 2, 1, 3)),
        sim[:, None, :], feat_indices[:, None, :])
    ref_indices = ridx.reshape(B, n_refs, num)

    y = fg.reshape(B, c, h, wd)
    for i, ((wgt, b), (s, p)) in enumerate(zip(
            dec, [(1, 1), (1, 1), (1, 1), (1, 1), (1, 2)])):
        if i < 4:
            y = jnp.repeat(jnp.repeat(y, 2, axis=2), 2, axis=3)
        last = i == 4
        y = _conv_block(y, wgt, b, s, p,
                        epilogue='tanh' if last else 'none')
        if not last:
            y = _bn_lrelu(y)

    return (y, sim, feat_indices, ref_indices, 0)
